# baseline (speedup 1.0000x reference)
import functools
import jax
import jax.numpy as jnp
from jax.experimental import pallas as pl
from jax.experimental.pallas import tpu as pltpu

K = 100000
D = 512
C = 64
NN = 64
BK = 2000
NB = K // BK
PW = 2048


def _knn_kernel(x_ref, q_ref, lab_ref, out_ref, d_ref, l_ref):
    b = pl.program_id(0)

    xb = x_ref[...]
    q = q_ref[...]
    diff = xb - q[None, :]
    sq = diff * diff
    ones_row = jnp.ones((1, D), dtype=jnp.float32)
    dsq = jax.lax.dot_general(
        ones_row, sq,
        dimension_numbers=(((1,), (1,)), ((), ())),
        precision=jax.lax.Precision.HIGHEST,
        preferred_element_type=jnp.float32,
    )
    dist = jnp.sqrt(dsq)

    @pl.when(b == 0)
    def _init():
        d_ref[...] = jnp.full((NB, PW), jnp.inf, dtype=jnp.float32)

    d_ref[pl.ds(b, 1), pl.ds(0, BK)] = dist
    l_ref[pl.ds(b, 1), pl.ds(0, BK)] = lab_ref[...].reshape(1, BK)

    @pl.when(b == NB - 1)
    def _finish():
        row_i = jax.lax.broadcasted_iota(jnp.int32, (NB, PW), 0)
        col_i = jax.lax.broadcasted_iota(jnp.int32, (NB, PW), 1)
        gidx = row_i * PW + col_i
        iota64 = jax.lax.broadcasted_iota(jnp.int32, (1, NN), 1)
        BIGI = jnp.int32(2147483647)

        def body(t, carry):
            d_top, lab_top = carry
            a = d_ref[...]
            m = jnp.min(a)
            idx = jnp.min(jnp.where(a == m, gidx, BIGI))
            hit = gidx == idx
            lab = jnp.max(jnp.where(hit, l_ref[...], -1))
            d_ref[...] = jnp.where(hit, jnp.inf, a)
            d_top = jnp.where(iota64 == t, m, d_top)
            lab_top = jnp.where(iota64 == t, lab, lab_top)
            return d_top, lab_top

        d_top0 = jnp.full((1, NN), jnp.inf, dtype=jnp.float32)
        lab_top0 = jnp.zeros((1, NN), dtype=jnp.int32)
        d_top, lab_top = jax.lax.fori_loop(0, NN, body, (d_top0, lab_top0))

        qv = jnp.float32(1.0) / (-d_top)
        amask = lab_top.reshape(NN, 1) == iota64
        S = jnp.sum(jnp.where(amask, qv, jnp.float32(0.0)),
                    axis=1, keepdims=True)
        rmax = jnp.max(S)
        iota_col = jax.lax.broadcasted_iota(jnp.int32, (NN, 1), 0)
        r = jnp.min(jnp.where(S == rmax, iota_col, jnp.int32(NN)))
        out_ref[...] = (iota64 == r).astype(jnp.float32).reshape(NN)


@jax.jit
def kernel(input, X, y):
    labels = jnp.argmax(y, axis=1).astype(jnp.int32).reshape(NB, 1, BK)
    out = pl.pallas_call(
        _knn_kernel,
        grid=(NB,),
        in_specs=[
            pl.BlockSpec((BK, D), lambda b: (b, 0)),
            pl.BlockSpec((D,), lambda b: (0,)),
            pl.BlockSpec((1, 1, BK), lambda b: (b, 0, 0)),
        ],
        out_specs=pl.BlockSpec((NN,), lambda b: (0,)),
        out_shape=jax.ShapeDtypeStruct((NN,), jnp.float32),
        scratch_shapes=[
            pltpu.VMEM((NB, PW), jnp.float32),
            pltpu.VMEM((NB, PW), jnp.int32),
        ],
    )(X, input, labels)
    return out


# speedup vs baseline: 1.7229x; 1.7229x over previous
import jax
import jax.numpy as jnp
from jax.experimental import pallas as pl
from jax.experimental.pallas import tpu as pltpu

K = 100000
D = 512
C = 64
NN = 64
BK = 2048
NB = (K + BK - 1) // BK
KP = NB * BK


def _knn_kernel(x_ref, q_ref, lab_ref, out_ref, d_ref, l_ref):
    b = pl.program_id(0)

    xb = x_ref[...]
    q = q_ref[...]
    diff = xb - q[None, :]
    sq = diff * diff
    part = (sq[:, 0:128] + sq[:, 128:256]) + (sq[:, 256:384] + sq[:, 384:512])
    pt = jax.lax.transpose(part, (1, 0))
    dsq = jnp.sum(pt, axis=0, keepdims=True)
    dist = jnp.sqrt(dsq)
    col = jax.lax.broadcasted_iota(jnp.int32, (1, BK), 1)
    valid = (b * BK + col) < K
    dist = jnp.where(valid, dist, jnp.inf)

    d_ref[pl.ds(b, 1), :] = dist
    l_ref[pl.ds(b, 1), :] = lab_ref[...].reshape(1, BK)

    @pl.when(b == NB - 1)
    def _finish():
        row_i = jax.lax.broadcasted_iota(jnp.int32, (NB, BK), 0)
        col_i = jax.lax.broadcasted_iota(jnp.int32, (NB, BK), 1)
        gidx = row_i * BK + col_i
        iota64 = jax.lax.broadcasted_iota(jnp.int32, (1, NN), 1)
        BIGI = jnp.int32(2147483647)

        def body(t, carry):
            d_top, lab_top = carry
            a = d_ref[...]
            m = jnp.min(a)
            idx = jnp.min(jnp.where(a == m, gidx, BIGI))
            hit = gidx == idx
            lab = jnp.max(jnp.where(hit, l_ref[...], -1))
            d_ref[...] = jnp.where(hit, jnp.inf, a)
            d_top = jnp.where(iota64 == t, m, d_top)
            lab_top = jnp.where(iota64 == t, lab, lab_top)
            return d_top, lab_top

        d_top0 = jnp.full((1, NN), jnp.inf, dtype=jnp.float32)
        lab_top0 = jnp.zeros((1, NN), dtype=jnp.int32)
        d_top, lab_top = jax.lax.fori_loop(0, NN, body, (d_top0, lab_top0))

        qv = jnp.float32(1.0) / (-d_top)
        amask = lab_top.reshape(NN, 1) == iota64
        S = jnp.sum(jnp.where(amask, qv, jnp.float32(0.0)),
                    axis=1, keepdims=True)
        rmax = jnp.max(S)
        iota_col = jax.lax.broadcasted_iota(jnp.int32, (NN, 1), 0)
        r = jnp.min(jnp.where(S == rmax, iota_col, jnp.int32(NN)))
        out_ref[...] = (iota64 == r).astype(jnp.float32).reshape(NN)


@jax.jit
def kernel(input, X, y):
    labels = jnp.argmax(y, axis=1).astype(jnp.int32)
    labels = jnp.pad(labels, (0, KP - K)).reshape(NB, 1, BK)
    out = pl.pallas_call(
        _knn_kernel,
        grid=(NB,),
        in_specs=[
            pl.BlockSpec((BK, D), lambda b: (b, 0)),
            pl.BlockSpec((D,), lambda b: (0,)),
            pl.BlockSpec((1, 1, BK), lambda b: (b, 0, 0)),
        ],
        out_specs=pl.BlockSpec((NN,), lambda b: (0,)),
        out_shape=jax.ShapeDtypeStruct((NN,), jnp.float32),
        scratch_shapes=[
            pltpu.VMEM((NB, BK), jnp.float32),
            pltpu.VMEM((NB, BK), jnp.int32),
        ],
    )(X, input, labels)
    return out
